# Initial kernel scaffold; baseline (speedup 1.0000x reference)
import jax
import jax.numpy as jnp
from jax.experimental import pallas as pl


def kernel(atomic_numbers, positions, edge_index, shifts, Wz, rW1, rb1, rW2, rb2, rW3, rb3, mixW, sc_w1, sc_w2):
    raise NotImplementedError("write your pallas kernel here")


# speedup vs baseline: 16.3664x; 16.3664x over previous
import functools

import jax
import jax.numpy as jnp
import numpy as np
from jax import lax
from jax.experimental import pallas as pl
from jax.experimental.pallas import tpu as pltpu
from jax.experimental.pallas import tpu_sc as plsc

N = 10000
E = 320000
K = 16
NRBF = 8
NELEM = 10
MAXL = 2
RCUT = 5.0
LM = (MAXL + 1) ** 2
ATTR = LM * K

NC = 2
NS = 16
NW = NC * NS
EPW = E // NW
EPT = E // NW
NPAD = 10240
NPT = NPAD // NS

EB = 2560
NEB = E // EB
NFB = 1280
SB = 200
NSC = EPT // SB


def _sc_gather_body(snd_hbm, rcv_hbm, px_hbm, py_hbm, pz_hbm, an_hbm,
                    vx_out, vy_out, vz_out, ans_out,
                    px_v, py_v, pz_v, an_v, snd_v, rcv_v,
                    vx_v, vy_v, vz_v, ans_v):
    c = lax.axis_index("c")
    s = lax.axis_index("s")
    wid = c * NS + s
    base = wid * EPW
    pltpu.sync_copy(px_hbm, px_v)
    pltpu.sync_copy(py_hbm, py_v)
    pltpu.sync_copy(pz_hbm, pz_v)
    pltpu.sync_copy(an_hbm, an_v)
    pltpu.sync_copy(snd_hbm.at[pl.ds(base, EPW)], snd_v)
    pltpu.sync_copy(rcv_hbm.at[pl.ds(base, EPW)], rcv_v)

    def body(i, carry):
        o = i * 16
        si = snd_v[pl.ds(o, 16)]
        ri = rcv_v[pl.ds(o, 16)]
        sx = plsc.load_gather(px_v, [si])
        sy = plsc.load_gather(py_v, [si])
        sz = plsc.load_gather(pz_v, [si])
        rx = plsc.load_gather(px_v, [ri])
        ry = plsc.load_gather(py_v, [ri])
        rz = plsc.load_gather(pz_v, [ri])
        za = plsc.load_gather(an_v, [si])
        vx_v[pl.ds(o, 16)] = rx - sx
        vy_v[pl.ds(o, 16)] = ry - sy
        vz_v[pl.ds(o, 16)] = rz - sz
        ans_v[pl.ds(o, 16)] = za
        return carry

    lax.fori_loop(0, EPW // 16, body, 0)
    pltpu.sync_copy(vx_v, vx_out.at[pl.ds(base, EPW)])
    pltpu.sync_copy(vy_v, vy_out.at[pl.ds(base, EPW)])
    pltpu.sync_copy(vz_v, vz_out.at[pl.ds(base, EPW)])
    pltpu.sync_copy(ans_v, ans_out.at[pl.ds(base, EPW)])


def _sc_gather(snd, rcv, px, py, pz, anf):
    mesh = plsc.VectorSubcoreMesh(core_axis_name="c", subcore_axis_name="s")
    f = pl.kernel(
        _sc_gather_body,
        out_type=[jax.ShapeDtypeStruct((E,), jnp.float32)] * 3
        + [jax.ShapeDtypeStruct((E,), jnp.float32)],
        mesh=mesh,
        scratch_types=[
            pltpu.VMEM((N,), jnp.float32),
            pltpu.VMEM((N,), jnp.float32),
            pltpu.VMEM((N,), jnp.float32),
            pltpu.VMEM((N,), jnp.float32),
            pltpu.VMEM((EPW,), jnp.int32),
            pltpu.VMEM((EPW,), jnp.int32),
            pltpu.VMEM((EPW,), jnp.float32),
            pltpu.VMEM((EPW,), jnp.float32),
            pltpu.VMEM((EPW,), jnp.float32),
            pltpu.VMEM((EPW,), jnp.float32),
        ],
        compiler_params=pltpu.CompilerParams(needs_layout_passes=False),
    )
    return f(snd, rcv, px, py, pz, anf)


def _tc_edge_body(vx_ref, vy_ref, vz_ref, shx_ref, shy_ref, shz_ref, ans_ref,
                  Wz_ref, W1_ref, b1_ref, W2_ref, b2_ref, W3_ref, b3_ref,
                  rb_ref, ylm_ref, attr_ref):
    f32 = jnp.float32
    x = vx_ref[0, 0] + shx_ref[0, 0]
    y = vy_ref[0, 0] + shy_ref[0, 0]
    z = vz_ref[0, 0] + shz_ref[0, 0]
    r2 = x * x + y * y + z * z + 1e-12
    r = jnp.sqrt(r2)
    inv_r = 1.0 / r
    ux = x * inv_r
    uy = y * inv_r
    uz = z * inv_r
    cutoff = 0.5 * (jnp.cos(jnp.pi * r / RCUT) + 1.0) * (r < RCUT).astype(f32)
    nvec = (lax.broadcasted_iota(jnp.int32, (1, NRBF), 1).astype(f32)
            + 1.0) * np.float32(np.pi / RCUT)
    bess = np.float32(np.sqrt(2.0 / RCUT)) * jnp.sin(
        nvec * r[:, None]) * inv_r[:, None]
    rbas = bess * cutoff[:, None]
    rb_ref[...] = rbas

    c1 = np.float32(3.0 ** 0.5)
    c2 = np.float32(15.0 ** 0.5)
    c3 = np.float32(0.5 * 5.0 ** 0.5)
    ylm = jnp.concatenate(
        [jnp.ones_like(ux)[:, None],
         (c1 * ux)[:, None], (c1 * uy)[:, None], (c1 * uz)[:, None],
         (c2 * ux * uy)[:, None], (c2 * uy * uz)[:, None],
         (c3 * (3.0 * uz * uz - 1.0))[:, None],
         (c2 * ux * uz)[:, None], (0.5 * c2 * (ux * ux - uy * uy))[:, None]],
        axis=1)
    ylm_ref[...] = ylm

    an = ans_ref[0, 0]
    oh = (an[:, None] ==
          lax.broadcasted_iota(jnp.int32, (EB, NELEM), 1).astype(f32)).astype(f32)
    zs = jnp.dot(oh, Wz_ref[...], preferred_element_type=f32)

    h = jnp.dot(rbas, W1_ref[...], preferred_element_type=f32) + b1_ref[0]
    h = h * jax.nn.sigmoid(h)
    h = jnp.dot(h, W2_ref[...], preferred_element_type=f32) + b2_ref[0]
    h = h * jax.nn.sigmoid(h)
    R = jnp.dot(h, W3_ref[...], preferred_element_type=f32) + b3_ref[0]
    g0 = R[:, 0:K] * zs
    g1 = R[:, K:2 * K] * zs
    g2 = R[:, 2 * K:3 * K] * zs
    pieces = []
    gl = [g0, g1, g1, g1, g2, g2, g2, g2, g2]
    for i in range(LM):
        pieces.append(ylm[:, i][:, None] * gl[i])
    attr_ref[...] = jnp.concatenate(pieces, axis=1)


def _tc_edge(vx, vy, vz, shx, shy, shz, ans, Wz, W1c, b1c, W2bd, b2c, W3bd, b3c):
    f32 = jnp.float32
    eb_spec = pl.BlockSpec((1, 1, EB), lambda i: (i, 0, 0))

    def wspec(shape):
        return pl.BlockSpec(shape, lambda i: tuple(0 for _ in shape))

    return pl.pallas_call(
        _tc_edge_body,
        grid=(NEB,),
        in_specs=[eb_spec] * 7 + [
            wspec((NELEM, K)), wspec((NRBF, 96)), wspec((1, 96)),
            wspec((96, 96)), wspec((1, 96)), wspec((96, 48)), wspec((1, 48)),
        ],
        out_specs=[
            pl.BlockSpec((EB, NRBF), lambda i: (i, 0)),
            pl.BlockSpec((EB, LM), lambda i: (i, 0)),
            pl.BlockSpec((EB, ATTR), lambda i: (i, 0)),
        ],
        out_shape=[
            jax.ShapeDtypeStruct((E, NRBF), f32),
            jax.ShapeDtypeStruct((E, LM), f32),
            jax.ShapeDtypeStruct((E, ATTR), f32),
        ],
    )(vx, vy, vz, shx, shy, shz, ans, Wz, W1c, b1c, W2bd, b2c, W3bd, b3c)


def _sc_scatter_body(attr_hbm, rcv_hbm, zeros_hbm, out0_hbm, out1_hbm,
                     acc_sh, rows_v, idx_v):
    c = lax.axis_index("c")
    s = lax.axis_index("s")
    pltpu.sync_copy(zeros_hbm.at[pl.ds(s * NPT, NPT)], acc_sh.at[pl.ds(s * NPT, NPT)])
    plsc.subcore_barrier()
    base = c * (E // NC) + s * EPT

    def body(j, carry):
        o = base + j * SB
        pltpu.sync_copy(attr_hbm.at[pl.ds(o, SB)], rows_v)
        pltpu.sync_copy(rcv_hbm.at[pl.ds(o, SB)], idx_v)
        pltpu.sync_copy(rows_v, acc_sh.at[idx_v], add=True)
        return carry

    lax.fori_loop(0, NSC, body, 0)
    plsc.subcore_barrier()

    @pl.when(c == 0)
    def _():
        pltpu.sync_copy(acc_sh.at[pl.ds(s * NPT, NPT)],
                        out0_hbm.at[pl.ds(s * NPT, NPT)])

    @pl.when(c == 1)
    def _():
        pltpu.sync_copy(acc_sh.at[pl.ds(s * NPT, NPT)],
                        out1_hbm.at[pl.ds(s * NPT, NPT)])


def _sc_scatter(attr, rcv, zeros_hbm):
    mesh = plsc.VectorSubcoreMesh(core_axis_name="c", subcore_axis_name="s")
    f = pl.kernel(
        _sc_scatter_body,
        out_type=[jax.ShapeDtypeStruct((NPAD, ATTR), jnp.float32)] * 2,
        mesh=mesh,
        scratch_types=[
            pltpu.VMEM_SHARED((NPAD, ATTR), jnp.float32),
            pltpu.VMEM((SB, ATTR), jnp.float32),
            pltpu.VMEM((SB,), jnp.int32),
        ],
        compiler_params=pltpu.CompilerParams(use_tc_tiling_on_sc=False),
    )
    return f(attr, rcv, zeros_hbm)


def _tc_finish_body(p0_ref, p1_ref, anf_ref, mixbd_ref, w1t_ref, w2t_ref, out_ref):
    f32 = jnp.float32
    nodeA = p0_ref[...] + p1_ref[...]
    A = jnp.dot(nodeA, mixbd_ref[...], preferred_element_type=f32)
    A0 = A[:, 0:K]
    s1 = A[:, K:2 * K] ** 2 + A[:, 2 * K:3 * K] ** 2 + A[:, 3 * K:4 * K] ** 2
    s2 = (A[:, 4 * K:5 * K] ** 2 + A[:, 5 * K:6 * K] ** 2 + A[:, 6 * K:7 * K] ** 2
          + A[:, 7 * K:8 * K] ** 2 + A[:, 8 * K:9 * K] ** 2)
    an = anf_ref[...][:, 0]
    oh = (an[:, None] ==
          lax.broadcasted_iota(jnp.int32, (NFB, NELEM), 1).astype(f32)).astype(f32)
    w1 = jnp.dot(oh, w1t_ref[...], preferred_element_type=f32)
    w2 = jnp.dot(oh, w2t_ref[...], preferred_element_type=f32)
    out_ref[...] = (w1 * A0 + w2[:, 0:K] * A0 * A0
                    + w2[:, K:2 * K] * (s1 * np.float32(1.0 / np.sqrt(3.0)))
                    + w2[:, 2 * K:3 * K] * (s2 * np.float32(1.0 / np.sqrt(5.0))))


def _tc_finish(p0, p1, anf2, mixbd, sc_w1, sc_w2f):
    f32 = jnp.float32

    def wspec(shape):
        return pl.BlockSpec(shape, lambda i: tuple(0 for _ in shape))

    return pl.pallas_call(
        _tc_finish_body,
        grid=(NPAD // NFB,),
        in_specs=[
            pl.BlockSpec((NFB, ATTR), lambda i: (i, 0)),
            pl.BlockSpec((NFB, ATTR), lambda i: (i, 0)),
            pl.BlockSpec((NFB, 1), lambda i: (i, 0)),
            wspec((ATTR, ATTR)), wspec((NELEM, K)), wspec((NELEM, 3 * K)),
        ],
        out_specs=pl.BlockSpec((NFB, K), lambda i: (i, 0)),
        out_shape=jax.ShapeDtypeStruct((NPAD, K), f32),
    )(p0, p1, anf2, mixbd, sc_w1, sc_w2f)


def kernel(atomic_numbers, positions, edge_index, shifts, Wz, rW1, rb1, rW2,
           rb2, rW3, rb3, mixW, sc_w1, sc_w2):
    f32 = jnp.float32
    snd = edge_index[0].astype(jnp.int32)
    rcv = edge_index[1].astype(jnp.int32)
    px = positions[:, 0].astype(f32)
    py = positions[:, 1].astype(f32)
    pz = positions[:, 2].astype(f32)
    anf = atomic_numbers.astype(f32)

    vx, vy, vz, ans = _sc_gather(snd, rcv, px, py, pz, anf)

    shx = shifts[:, 0].reshape(NEB, 1, EB).astype(f32)
    shy = shifts[:, 1].reshape(NEB, 1, EB).astype(f32)
    shz = shifts[:, 2].reshape(NEB, 1, EB).astype(f32)

    W1c = jnp.concatenate([rW1[l] for l in range(MAXL + 1)], axis=1)
    b1c = jnp.concatenate([rb1[l] for l in range(MAXL + 1)], axis=0)[None]
    Z32 = jnp.zeros((32, 32), f32)
    W2bd = jnp.block([[rW2[0], Z32, Z32], [Z32, rW2[1], Z32], [Z32, Z32, rW2[2]]])
    b2c = jnp.concatenate([rb2[l] for l in range(MAXL + 1)], axis=0)[None]
    Z3K = jnp.zeros((32, K), f32)
    W3bd = jnp.block([[rW3[0], Z3K, Z3K], [Z3K, rW3[1], Z3K], [Z3K, Z3K, rW3[2]]])
    b3c = jnp.concatenate([rb3[l] for l in range(MAXL + 1)], axis=0)[None]

    rb, ylm, attr = _tc_edge(
        vx.reshape(NEB, 1, EB), vy.reshape(NEB, 1, EB), vz.reshape(NEB, 1, EB),
        shx, shy, shz, ans.reshape(NEB, 1, EB),
        Wz, W1c, b1c, W2bd, b2c, W3bd, b3c)

    zeros_hbm = jnp.zeros((NPAD, ATTR), f32)
    p0, p1 = _sc_scatter(attr, rcv, zeros_hbm)

    lmap = [0, 1, 1, 1, 2, 2, 2, 2, 2]
    scale = np.float32(1.0 / np.sqrt(float(K)))
    ZKK = jnp.zeros((K, K), f32)
    mixbd = jnp.block([[mixW[lmap[i]] * scale if i == j else ZKK
                        for j in range(LM)] for i in range(LM)])

    anf2 = jnp.concatenate([anf, jnp.zeros((NPAD - N,), f32)])[:, None]
    sc_w2f = sc_w2.reshape(NELEM, 3 * K)
    B = _tc_finish(p0, p1, anf2, mixbd, sc_w1, sc_w2f)
    return (B[:N], rb, ylm)


# speedup vs baseline: 26.8674x; 1.6416x over previous
import functools

import jax
import jax.numpy as jnp
import numpy as np
from jax import lax
from jax.experimental import pallas as pl
from jax.experimental.pallas import tpu as pltpu
from jax.experimental.pallas import tpu_sc as plsc

N = 10000
E = 320000
K = 16
NRBF = 8
NELEM = 10
MAXL = 2
RCUT = 5.0
LM = (MAXL + 1) ** 2
ATTR = LM * K

NC = 2
NS = 16
NW = NC * NS
EPW = E // NW
EPT = E // NW
NPAD = 10240
NPT = NPAD // NS

EB = 2560
NEB = E // EB
NFB = 1280
SB = 200
NSC = EPT // SB


def _sc_gather_body(snd_hbm, rcv_hbm, px_hbm, py_hbm, pz_hbm, an_hbm,
                    vx_out, vy_out, vz_out, ans_out,
                    px_v, py_v, pz_v, an_v, snd_v, rcv_v,
                    vx_v, vy_v, vz_v, ans_v):
    c = lax.axis_index("c")
    s = lax.axis_index("s")
    wid = c * NS + s
    base = wid * EPW
    pltpu.sync_copy(px_hbm, px_v)
    pltpu.sync_copy(py_hbm, py_v)
    pltpu.sync_copy(pz_hbm, pz_v)
    pltpu.sync_copy(an_hbm, an_v)
    pltpu.sync_copy(snd_hbm.at[pl.ds(base, EPW)], snd_v)
    pltpu.sync_copy(rcv_hbm.at[pl.ds(base, EPW)], rcv_v)

    def body(i, carry):
        o = i * 16
        si = snd_v[pl.ds(o, 16)]
        ri = rcv_v[pl.ds(o, 16)]
        sx = plsc.load_gather(px_v, [si])
        sy = plsc.load_gather(py_v, [si])
        sz = plsc.load_gather(pz_v, [si])
        rx = plsc.load_gather(px_v, [ri])
        ry = plsc.load_gather(py_v, [ri])
        rz = plsc.load_gather(pz_v, [ri])
        za = plsc.load_gather(an_v, [si])
        vx_v[pl.ds(o, 16)] = rx - sx
        vy_v[pl.ds(o, 16)] = ry - sy
        vz_v[pl.ds(o, 16)] = rz - sz
        ans_v[pl.ds(o, 16)] = za
        return carry

    lax.fori_loop(0, EPW // 16, body, 0)
    pltpu.sync_copy(vx_v, vx_out.at[pl.ds(base, EPW)])
    pltpu.sync_copy(vy_v, vy_out.at[pl.ds(base, EPW)])
    pltpu.sync_copy(vz_v, vz_out.at[pl.ds(base, EPW)])
    pltpu.sync_copy(ans_v, ans_out.at[pl.ds(base, EPW)])


def _sc_gather(snd, rcv, px, py, pz, anf):
    mesh = plsc.VectorSubcoreMesh(core_axis_name="c", subcore_axis_name="s")
    f = pl.kernel(
        _sc_gather_body,
        out_type=[jax.ShapeDtypeStruct((E,), jnp.float32)] * 3
        + [jax.ShapeDtypeStruct((E,), jnp.float32)],
        mesh=mesh,
        scratch_types=[
            pltpu.VMEM((N,), jnp.float32),
            pltpu.VMEM((N,), jnp.float32),
            pltpu.VMEM((N,), jnp.float32),
            pltpu.VMEM((N,), jnp.float32),
            pltpu.VMEM((EPW,), jnp.int32),
            pltpu.VMEM((EPW,), jnp.int32),
            pltpu.VMEM((EPW,), jnp.float32),
            pltpu.VMEM((EPW,), jnp.float32),
            pltpu.VMEM((EPW,), jnp.float32),
            pltpu.VMEM((EPW,), jnp.float32),
        ],
        compiler_params=pltpu.CompilerParams(needs_layout_passes=False),
    )
    return f(snd, rcv, px, py, pz, anf)


def _tc_edge_body(vx_ref, vy_ref, vz_ref, shx_ref, shy_ref, shz_ref, ans_ref,
                  Wz_ref, W1_ref, b1_ref, W2_ref, b2_ref, W3_ref, b3_ref,
                  M1_ref, M2_ref, dY_ref, EXP9_ref, EXP48_ref, TILE3_ref,
                  rb_ref, ylm_ref, attr_ref):
    f32 = jnp.float32
    x = vx_ref[0, 0] + shx_ref[0, 0]
    y = vy_ref[0, 0] + shy_ref[0, 0]
    z = vz_ref[0, 0] + shz_ref[0, 0]
    r2 = x * x + y * y + z * z + 1e-12
    r = jnp.sqrt(r2)
    inv_r = 1.0 / r
    cutoff = 0.5 * (jnp.cos(jnp.pi * r / RCUT) + 1.0) * (r < RCUT).astype(f32)
    nvec = (lax.broadcasted_iota(jnp.int32, (1, NRBF), 1).astype(f32)
            + 1.0) * np.float32(np.pi / RCUT)
    bess = np.float32(np.sqrt(2.0 / RCUT)) * jnp.sin(
        nvec * r[:, None]) * inv_r[:, None]
    rbas = bess * cutoff[:, None]
    rb_ref[...] = rbas

    u4 = jnp.concatenate(
        [jnp.ones_like(x)[:, None], (x * inv_r)[:, None],
         (y * inv_r)[:, None], (z * inv_r)[:, None]], axis=1)
    ylm = (jnp.dot(u4, M1_ref[...], preferred_element_type=f32)
           * jnp.dot(u4, M2_ref[...], preferred_element_type=f32)
           + dY_ref[0])
    ylm_ref[...] = ylm

    an = ans_ref[0, 0]
    oh = (an[:, None] ==
          lax.broadcasted_iota(jnp.int32, (EB, NELEM), 1).astype(f32)).astype(f32)
    zs = jnp.dot(oh, Wz_ref[...], preferred_element_type=f32)

    h = jnp.dot(rbas, W1_ref[...], preferred_element_type=f32) + b1_ref[0]
    h = h * jax.nn.sigmoid(h)
    h = jnp.dot(h, W2_ref[...], preferred_element_type=f32) + b2_ref[0]
    h = h * jax.nn.sigmoid(h)
    R = jnp.dot(h, W3_ref[...], preferred_element_type=f32) + b3_ref[0]
    G = R * jnp.dot(zs, TILE3_ref[...], preferred_element_type=f32)
    attr_ref[...] = (jnp.dot(ylm, EXP9_ref[...], preferred_element_type=f32)
                     * jnp.dot(G, EXP48_ref[...], preferred_element_type=f32))


def _tc_edge(vx, vy, vz, shx, shy, shz, ans, Wz, W1c, b1c, W2bd, b2c, W3bd,
             b3c, M1, M2, dY, EXP9, EXP48, TILE3):
    f32 = jnp.float32
    eb_spec = pl.BlockSpec((1, 1, EB), lambda i: (i, 0, 0))

    def wspec(shape):
        return pl.BlockSpec(shape, lambda i: tuple(0 for _ in shape))

    return pl.pallas_call(
        _tc_edge_body,
        grid=(NEB,),
        in_specs=[eb_spec] * 7 + [
            wspec((NELEM, K)), wspec((NRBF, 96)), wspec((1, 96)),
            wspec((96, 96)), wspec((1, 96)), wspec((96, 48)), wspec((1, 48)),
            wspec((4, LM)), wspec((4, LM)), wspec((1, LM)),
            wspec((LM, ATTR)), wspec((3 * K, ATTR)), wspec((K, 3 * K)),
        ],
        out_specs=[
            pl.BlockSpec((EB, NRBF), lambda i: (i, 0)),
            pl.BlockSpec((EB, LM), lambda i: (i, 0)),
            pl.BlockSpec((EB, ATTR), lambda i: (i, 0)),
        ],
        out_shape=[
            jax.ShapeDtypeStruct((E, NRBF), f32),
            jax.ShapeDtypeStruct((E, LM), f32),
            jax.ShapeDtypeStruct((E, ATTR), f32),
        ],
    )(vx, vy, vz, shx, shy, shz, ans, Wz, W1c, b1c, W2bd, b2c, W3bd, b3c,
      M1, M2, dY, EXP9, EXP48, TILE3)


def _sc_scatter_body(attr_hbm, rcv_hbm, zeros_hbm, out0_hbm, out1_hbm,
                     acc_sh, rows_v, idx_v):
    c = lax.axis_index("c")
    s = lax.axis_index("s")
    pltpu.sync_copy(zeros_hbm.at[pl.ds(s * NPT, NPT)], acc_sh.at[pl.ds(s * NPT, NPT)])
    plsc.subcore_barrier()
    base = c * (E // NC) + s * EPT

    def body(j, carry):
        o = base + j * SB
        pltpu.sync_copy(attr_hbm.at[pl.ds(o, SB)], rows_v)
        pltpu.sync_copy(rcv_hbm.at[pl.ds(o, SB)], idx_v)
        pltpu.sync_copy(rows_v, acc_sh.at[idx_v], add=True)
        return carry

    lax.fori_loop(0, NSC, body, 0)
    plsc.subcore_barrier()

    @pl.when(c == 0)
    def _():
        pltpu.sync_copy(acc_sh.at[pl.ds(s * NPT, NPT)],
                        out0_hbm.at[pl.ds(s * NPT, NPT)])

    @pl.when(c == 1)
    def _():
        pltpu.sync_copy(acc_sh.at[pl.ds(s * NPT, NPT)],
                        out1_hbm.at[pl.ds(s * NPT, NPT)])


def _sc_scatter(attr, rcv, zeros_hbm):
    mesh = plsc.VectorSubcoreMesh(core_axis_name="c", subcore_axis_name="s")
    f = pl.kernel(
        _sc_scatter_body,
        out_type=[jax.ShapeDtypeStruct((NPAD, ATTR), jnp.float32)] * 2,
        mesh=mesh,
        scratch_types=[
            pltpu.VMEM_SHARED((NPAD, ATTR), jnp.float32),
            pltpu.VMEM((SB, ATTR), jnp.float32),
            pltpu.VMEM((SB,), jnp.int32),
        ],
        compiler_params=pltpu.CompilerParams(use_tc_tiling_on_sc=False),
    )
    return f(attr, rcv, zeros_hbm)


def _tc_finish_body(p0_ref, p1_ref, anf_ref, mixbd_ref, w1t_ref, w2t_ref, out_ref):
    f32 = jnp.float32
    nodeA = p0_ref[...] + p1_ref[...]
    A = jnp.dot(nodeA, mixbd_ref[...], preferred_element_type=f32)
    A0 = A[:, 0:K]
    s1 = A[:, K:2 * K] ** 2 + A[:, 2 * K:3 * K] ** 2 + A[:, 3 * K:4 * K] ** 2
    s2 = (A[:, 4 * K:5 * K] ** 2 + A[:, 5 * K:6 * K] ** 2 + A[:, 6 * K:7 * K] ** 2
          + A[:, 7 * K:8 * K] ** 2 + A[:, 8 * K:9 * K] ** 2)
    an = anf_ref[...][:, 0]
    oh = (an[:, None] ==
          lax.broadcasted_iota(jnp.int32, (NFB, NELEM), 1).astype(f32)).astype(f32)
    w1 = jnp.dot(oh, w1t_ref[...], preferred_element_type=f32)
    w2 = jnp.dot(oh, w2t_ref[...], preferred_element_type=f32)
    out_ref[...] = (w1 * A0 + w2[:, 0:K] * A0 * A0
                    + w2[:, K:2 * K] * (s1 * np.float32(1.0 / np.sqrt(3.0)))
                    + w2[:, 2 * K:3 * K] * (s2 * np.float32(1.0 / np.sqrt(5.0))))


def _tc_finish(p0, p1, anf2, mixbd, sc_w1, sc_w2f):
    f32 = jnp.float32

    def wspec(shape):
        return pl.BlockSpec(shape, lambda i: tuple(0 for _ in shape))

    return pl.pallas_call(
        _tc_finish_body,
        grid=(NPAD // NFB,),
        in_specs=[
            pl.BlockSpec((NFB, ATTR), lambda i: (i, 0)),
            pl.BlockSpec((NFB, ATTR), lambda i: (i, 0)),
            pl.BlockSpec((NFB, 1), lambda i: (i, 0)),
            wspec((ATTR, ATTR)), wspec((NELEM, K)), wspec((NELEM, 3 * K)),
        ],
        out_specs=pl.BlockSpec((NFB, K), lambda i: (i, 0)),
        out_shape=jax.ShapeDtypeStruct((NPAD, K), f32),
    )(p0, p1, anf2, mixbd, sc_w1, sc_w2f)


def kernel(atomic_numbers, positions, edge_index, shifts, Wz, rW1, rb1, rW2,
           rb2, rW3, rb3, mixW, sc_w1, sc_w2):
    f32 = jnp.float32
    snd = edge_index[0].astype(jnp.int32)
    rcv = edge_index[1].astype(jnp.int32)
    px = positions[:, 0].astype(f32)
    py = positions[:, 1].astype(f32)
    pz = positions[:, 2].astype(f32)
    anf = atomic_numbers.astype(f32)

    vx, vy, vz, ans = _sc_gather(snd, rcv, px, py, pz, anf)

    shx = shifts[:, 0].reshape(NEB, 1, EB).astype(f32)
    shy = shifts[:, 1].reshape(NEB, 1, EB).astype(f32)
    shz = shifts[:, 2].reshape(NEB, 1, EB).astype(f32)

    W1c = jnp.concatenate([rW1[l] for l in range(MAXL + 1)], axis=1)
    b1c = jnp.concatenate([rb1[l] for l in range(MAXL + 1)], axis=0)[None]
    Z32 = jnp.zeros((32, 32), f32)
    W2bd = jnp.block([[rW2[0], Z32, Z32], [Z32, rW2[1], Z32], [Z32, Z32, rW2[2]]])
    b2c = jnp.concatenate([rb2[l] for l in range(MAXL + 1)], axis=0)[None]
    Z3K = jnp.zeros((32, K), f32)
    W3bd = jnp.block([[rW3[0], Z3K, Z3K], [Z3K, rW3[1], Z3K], [Z3K, Z3K, rW3[2]]])
    b3c = jnp.concatenate([rb3[l] for l in range(MAXL + 1)], axis=0)[None]

    c1 = np.sqrt(3.0)
    c2 = np.sqrt(15.0)
    c3 = 0.5 * np.sqrt(5.0)
    M1n = np.zeros((4, LM), np.float32)
    M2n = np.zeros((4, LM), np.float32)
    dYn = np.zeros((1, LM), np.float32)
    M1n[0, 0] = 1.0; M2n[0, 0] = 1.0
    M1n[1, 1] = c1;  M2n[0, 1] = 1.0
    M1n[2, 2] = c1;  M2n[0, 2] = 1.0
    M1n[3, 3] = c1;  M2n[0, 3] = 1.0
    M1n[1, 4] = c2;  M2n[2, 4] = 1.0
    M1n[2, 5] = c2;  M2n[3, 5] = 1.0
    M1n[3, 6] = 3.0 * c3; M2n[3, 6] = 1.0; dYn[0, 6] = -c3
    M1n[1, 7] = c2;  M2n[3, 7] = 1.0
    M1n[1, 8] = 0.5 * c2; M1n[2, 8] = 0.5 * c2
    M2n[1, 8] = 1.0; M2n[2, 8] = -1.0
    lmap = [0, 1, 1, 1, 2, 2, 2, 2, 2]
    EXP9n = np.zeros((LM, ATTR), np.float32)
    EXP48n = np.zeros((3 * K, ATTR), np.float32)
    TILE3n = np.zeros((K, 3 * K), np.float32)
    for i in range(LM):
        EXP9n[i, i * K:(i + 1) * K] = 1.0
        for k in range(K):
            EXP48n[lmap[i] * K + k, i * K + k] = 1.0
    for j in range(3):
        for k in range(K):
            TILE3n[k, j * K + k] = 1.0

    rb, ylm, attr = _tc_edge(
        vx.reshape(NEB, 1, EB), vy.reshape(NEB, 1, EB), vz.reshape(NEB, 1, EB),
        shx, shy, shz, ans.reshape(NEB, 1, EB),
        Wz, W1c, b1c, W2bd, b2c, W3bd, b3c,
        jnp.asarray(M1n), jnp.asarray(M2n), jnp.asarray(dYn),
        jnp.asarray(EXP9n), jnp.asarray(EXP48n), jnp.asarray(TILE3n))

    zeros_hbm = jnp.zeros((NPAD, ATTR), f32)
    p0, p1 = _sc_scatter(attr, rcv, zeros_hbm)

    scale = np.float32(1.0 / np.sqrt(float(K)))
    ZKK = jnp.zeros((K, K), f32)
    mixbd = jnp.block([[mixW[lmap[i]] * scale if i == j else ZKK
                        for j in range(LM)] for i in range(LM)])

    anf2 = jnp.concatenate([anf, jnp.zeros((NPAD - N,), f32)])[:, None]
    sc_w2f = sc_w2.reshape(NELEM, 3 * K)
    B = _tc_finish(p0, p1, anf2, mixbd, sc_w1, sc_w2f)
    return (B[:N], rb, ylm)
